# baseline (speedup 1.0000x reference)
import functools

import jax
import jax.numpy as jnp
from jax import lax
from jax.experimental import pallas as pl
from jax.experimental.pallas import tpu as pltpu
from jax.experimental.pallas import tpu_sc as plsc

_R = 32
_NW = 32
_LANES = 16
_UNROLL = 8


def kernel(x, pe_table):
    B, S, D = x.shape
    rows_per_w = S // _NW
    n_chunks = rows_per_w // _R
    chunk_words = _R * D

    xf = x.reshape(B * S * D)
    pef = pe_table[:S].reshape(S * D)

    mesh = plsc.VectorSubcoreMesh(core_axis_name="c", subcore_axis_name="s")

    @functools.partial(
        pl.kernel,
        mesh=mesh,
        out_type=jax.ShapeDtypeStruct((B * S * D,), jnp.float32),
        scratch_types=[
            pltpu.VMEM((chunk_words,), jnp.float32),
            pltpu.VMEM((chunk_words,), jnp.float32),
        ],
    )
    def sc_add(x_hbm, pe_hbm, out_hbm, pe_buf, x_buf):
        wid = lax.axis_index("s") * 2 + lax.axis_index("c")
        base = wid * rows_per_w * D

        def chunk_body(g, _):
            off = base + g * chunk_words
            pltpu.sync_copy(pe_hbm.at[pl.ds(off, chunk_words)], pe_buf)
            for b in range(B):
                xoff = b * S * D + off
                pltpu.sync_copy(x_hbm.at[pl.ds(xoff, chunk_words)], x_buf)

                def add_body(i, _):
                    j = i * (_LANES * _UNROLL)
                    for k in range(_UNROLL):
                        o = j + k * _LANES
                        x_buf[pl.ds(o, _LANES)] = (
                            x_buf[pl.ds(o, _LANES)] + pe_buf[pl.ds(o, _LANES)]
                        )
                    return 0

                lax.fori_loop(0, chunk_words // (_LANES * _UNROLL), add_body, 0)
                pltpu.sync_copy(x_buf, out_hbm.at[pl.ds(xoff, chunk_words)])
            return 0

        lax.fori_loop(0, n_chunks, chunk_body, 0)

    out = sc_add(xf, pef)
    return out.reshape(B, S, D)


# speedup vs baseline: 1.1898x; 1.1898x over previous
import functools

import jax
import jax.numpy as jnp
from jax import lax
from jax.experimental import pallas as pl
from jax.experimental.pallas import tpu as pltpu
from jax.experimental.pallas import tpu_sc as plsc

_R = 16
_NW = 32
_LANES = 16
_UNROLL = 8


def kernel(x, pe_table):
    B, S, D = x.shape
    rows_per_w = S // _NW
    n_chunks = rows_per_w // _R
    cw = _R * D
    n_blocks = n_chunks // 2

    xf = x.reshape(B * S * D)
    pef = pe_table[:S].reshape(S * D)

    mesh = plsc.VectorSubcoreMesh(core_axis_name="c", subcore_axis_name="s")

    @functools.partial(
        pl.kernel,
        mesh=mesh,
        out_type=jax.ShapeDtypeStruct((B * S * D,), jnp.float32),
        scratch_types=[
            pltpu.VMEM((cw,), jnp.float32),
            pltpu.VMEM((cw,), jnp.float32),
            pltpu.VMEM((cw,), jnp.float32),
            pltpu.VMEM((cw,), jnp.float32),
            pltpu.SemaphoreType.DMA,
            pltpu.SemaphoreType.DMA,
            pltpu.SemaphoreType.DMA,
            pltpu.SemaphoreType.DMA,
            pltpu.SemaphoreType.DMA,
            pltpu.SemaphoreType.DMA,
        ],
    )
    def sc_add(x_hbm, pe_hbm, out_hbm, pb0, pb1, xb0, xb1,
               ps0, ps1, gs0, gs1, ss0, ss1):
        pebufs = (pb0, pb1)
        xbufs = (xb0, xb1)
        psems = (ps0, ps1)
        gsems = (gs0, gs1)
        ssems = (ss0, ss1)

        wid = lax.axis_index("s") * 2 + lax.axis_index("c")
        base = wid * rows_per_w * D
        SD = S * D

        def pe_slice(g):
            return pe_hbm.at[pl.ds(base + g * cw, cw)]

        def x_slice(g, b):
            return x_hbm.at[pl.ds(b * SD + base + g * cw, cw)]

        def out_slice(g, b):
            return out_hbm.at[pl.ds(b * SD + base + g * cw, cw)]

        pltpu.async_copy(pe_slice(0), pb0, ps0)
        pltpu.async_copy(pe_slice(1), pb1, ps1)
        pltpu.async_copy(x_slice(0, 0), xb0, gs0)

        def add_chunk(xbuf, pebuf):
            def add_body(i, _):
                j = i * (_LANES * _UNROLL)
                for k in range(_UNROLL):
                    o = j + k * _LANES
                    xbuf[pl.ds(o, _LANES)] = (
                        xbuf[pl.ds(o, _LANES)] + pebuf[pl.ds(o, _LANES)]
                    )
                return 0

            lax.fori_loop(0, cw // (_LANES * _UNROLL), add_body, 0)

        def block_body(gg, _):
            for s in range(8):
                gp, b = divmod(s, 4)
                g = gg * 2 + gp
                xbuf = xbufs[s % 2]
                pebuf = pebufs[gp]

                pltpu.make_async_copy(x_slice(g, b), xbuf, gsems[s % 2]).wait()

                if s < 7:
                    ng, nb = divmod(s + 1, 4)
                    nxt = (gg * 2 + ng, nb)

                    @pl.when(jnp.logical_or(gg > 0, s > 0))
                    def _():
                        pltpu.make_async_copy(
                            out_slice(*_prev[0]), xbufs[(s + 1) % 2],
                            ssems[(s + 1) % 2]).wait()

                    pltpu.async_copy(x_slice(*nxt), xbufs[(s + 1) % 2],
                                     gsems[(s + 1) % 2])
                else:
                    @pl.when(gg < n_blocks - 1)
                    def _():
                        pltpu.make_async_copy(
                            out_slice(*_prev[0]), xbufs[0], ssems[0]).wait()
                        pltpu.async_copy(x_slice(gg * 2 + 2, 0), xbufs[0],
                                         gsems[0])

                if b == 0:
                    pltpu.make_async_copy(pe_slice(g), pebuf, psems[gp]).wait()

                add_chunk(xbuf, pebuf)

                if b == 3:
                    @pl.when(g + 2 < n_chunks)
                    def _():
                        pltpu.async_copy(pe_slice(g + 2), pebuf, psems[gp])

                pltpu.async_copy(xbuf, out_slice(g, b), ssems[s % 2])
                _prev[0] = (g, b)
            return 0

        _prev = [(0, 0)]
        lax.fori_loop(0, n_blocks, block_body, 0)

        pltpu.make_async_copy(out_slice(n_chunks - 1, 2), xb0, ss0).wait()
        pltpu.make_async_copy(out_slice(n_chunks - 1, 3), xb1, ss1).wait()

    out = sc_add(xf, pef)
    return out.reshape(B, S, D)


# speedup vs baseline: 3.3564x; 2.8209x over previous
import functools

import jax
import jax.numpy as jnp
from jax import lax
from jax.experimental import pallas as pl
from jax.experimental.pallas import tpu as pltpu
from jax.experimental.pallas import tpu_sc as plsc

_R = 16
_NW = 32
_LANES = 16
_UNROLL = 8


def kernel(x, pe_table):
    B, S, D = x.shape
    rows_per_w = S // _NW
    n_chunks = rows_per_w // _R
    n_blocks = n_chunks // 2

    pe = pe_table[:S]

    mesh = plsc.VectorSubcoreMesh(core_axis_name="c", subcore_axis_name="s")

    @functools.partial(
        pl.kernel,
        mesh=mesh,
        out_type=jax.ShapeDtypeStruct((B, S, D), jnp.float32),
        scratch_types=[
            pltpu.VMEM((_R, D), jnp.float32),
            pltpu.VMEM((_R, D), jnp.float32),
            pltpu.VMEM((_R, D), jnp.float32),
            pltpu.VMEM((_R, D), jnp.float32),
            pltpu.SemaphoreType.DMA,
            pltpu.SemaphoreType.DMA,
            pltpu.SemaphoreType.DMA,
            pltpu.SemaphoreType.DMA,
            pltpu.SemaphoreType.DMA,
            pltpu.SemaphoreType.DMA,
        ],
        compiler_params=pltpu.CompilerParams(use_tc_tiling_on_sc=True),
    )
    def sc_add(x_hbm, pe_hbm, out_hbm, pb0, pb1, xb0, xb1,
               ps0, ps1, gs0, gs1, ss0, ss1):
        pebufs = (pb0, pb1)
        xbufs = (xb0, xb1)
        psems = (ps0, ps1)
        gsems = (gs0, gs1)
        ssems = (ss0, ss1)

        wid = lax.axis_index("s") * 2 + lax.axis_index("c")
        base = wid * rows_per_w

        def pe_slice(g):
            return pe_hbm.at[pl.ds(base + g * _R, _R)]

        def x_slice(g, b):
            return x_hbm.at[b, pl.ds(base + g * _R, _R)]

        def out_slice(g, b):
            return out_hbm.at[b, pl.ds(base + g * _R, _R)]

        pltpu.async_copy(pe_slice(0), pb0, ps0)
        pltpu.async_copy(pe_slice(1), pb1, ps1)
        pltpu.async_copy(x_slice(0, 0), xb0, gs0)

        def add_chunk(xbuf, pebuf):
            def add_body(i, _):
                r = i >> 3
                cb = (i & 7) * (D // 8)
                for k in range(_UNROLL):
                    o = cb + k * _LANES
                    xbuf[r, pl.ds(o, _LANES)] = (
                        xbuf[r, pl.ds(o, _LANES)] + pebuf[r, pl.ds(o, _LANES)]
                    )
                return 0

            lax.fori_loop(0, _R * D // (_LANES * _UNROLL), add_body, 0)

        def block_body(gg, _):
            for s in range(8):
                gp, b = divmod(s, 4)
                g = gg * 2 + gp
                xbuf = xbufs[s % 2]
                pebuf = pebufs[gp]

                pltpu.make_async_copy(x_slice(g, b), xbuf, gsems[s % 2]).wait()

                if s < 7:
                    ng, nb = divmod(s + 1, 4)
                    nxt = (gg * 2 + ng, nb)

                    @pl.when(jnp.logical_or(gg > 0, s > 0))
                    def _():
                        pltpu.make_async_copy(
                            out_slice(*_prev[0]), xbufs[(s + 1) % 2],
                            ssems[(s + 1) % 2]).wait()

                    pltpu.async_copy(x_slice(*nxt), xbufs[(s + 1) % 2],
                                     gsems[(s + 1) % 2])
                else:
                    @pl.when(gg < n_blocks - 1)
                    def _():
                        pltpu.make_async_copy(
                            out_slice(*_prev[0]), xbufs[0], ssems[0]).wait()
                        pltpu.async_copy(x_slice(gg * 2 + 2, 0), xbufs[0],
                                         gsems[0])

                if b == 0:
                    pltpu.make_async_copy(pe_slice(g), pebuf, psems[gp]).wait()

                add_chunk(xbuf, pebuf)

                if b == 3:
                    @pl.when(g + 2 < n_chunks)
                    def _():
                        pltpu.async_copy(pe_slice(g + 2), pebuf, psems[gp])

                pltpu.async_copy(xbuf, out_slice(g, b), ssems[s % 2])
                _prev[0] = (g, b)
            return 0

        _prev = [(0, 0)]
        lax.fori_loop(0, n_blocks, block_body, 0)

        pltpu.make_async_copy(out_slice(n_chunks - 1, 2), xb0, ss0).wait()
        pltpu.make_async_copy(out_slice(n_chunks - 1, 3), xb1, ss1).wait()

    return sc_add(x, pe)


# speedup vs baseline: 3.6098x; 1.0755x over previous
import functools

import jax
import jax.numpy as jnp
from jax import lax
from jax.experimental import pallas as pl
from jax.experimental.pallas import tpu as pltpu
from jax.experimental.pallas import tpu_sc as plsc

_R = 8
_NW = 32
_LANES = 16
_B = 4


def kernel(x, pe_table):
    B, S, D = x.shape
    rows_per_w = S // _NW
    n_chunks = rows_per_w // _R
    n_blocks = n_chunks // 2

    pe = pe_table[:S]

    mesh = plsc.VectorSubcoreMesh(core_axis_name="c", subcore_axis_name="s")

    vmem = lambda: pltpu.VMEM((_R, D), jnp.float32)
    sem = pltpu.SemaphoreType.DMA

    @functools.partial(
        pl.kernel,
        mesh=mesh,
        out_type=jax.ShapeDtypeStruct((B, S, D), jnp.float32),
        scratch_types=(
            [vmem() for _ in range(2)]
            + [vmem() for _ in range(2 * _B)]
            + [sem] * 6
        ),
        compiler_params=pltpu.CompilerParams(use_tc_tiling_on_sc=True),
    )
    def sc_add(x_hbm, pe_hbm, out_hbm, pb0, pb1,
               xa0, xa1, xa2, xa3, xb0, xb1, xb2, xb3,
               ps0, ps1, gs0, gs1, ss0, ss1):
        pebufs = (pb0, pb1)
        xbufs = ((xa0, xa1, xa2, xa3), (xb0, xb1, xb2, xb3))
        psems = (ps0, ps1)
        gsems = (gs0, gs1)
        ssems = (ss0, ss1)

        wid = lax.axis_index("s") * 2 + lax.axis_index("c")
        base = wid * rows_per_w

        def pe_slice(g):
            return pe_hbm.at[pl.ds(base + g * _R, _R)]

        def x_slice(g, b):
            return x_hbm.at[b, pl.ds(base + g * _R, _R)]

        def out_slice(g, b):
            return out_hbm.at[b, pl.ds(base + g * _R, _R)]

        def issue_gathers(g, par):
            for b in range(_B):
                pltpu.async_copy(x_slice(g, b), xbufs[par][b], gsems[par])

        def wait_gathers(g, par):
            for b in range(_B):
                pltpu.make_async_copy(
                    x_slice(g, b), xbufs[par][b], gsems[par]).wait()

        def wait_scatters(g, par):
            for b in range(_B):
                pltpu.make_async_copy(
                    out_slice(g, b), xbufs[par][b], ssems[par]).wait()

        def issue_scatters(g, par):
            for b in range(_B):
                pltpu.async_copy(xbufs[par][b], out_slice(g, b), ssems[par])

        pltpu.async_copy(pe_slice(0), pb0, ps0)
        pltpu.async_copy(pe_slice(1), pb1, ps1)
        issue_gathers(0, 0)

        def add_chunk(par):
            bufs = xbufs[par]
            peb = pebufs[par]

            def add_body(i, _):
                r = i >> 3
                cb = (i & 7) * (D // 8)
                for k in range(8):
                    o = cb + k * _LANES
                    vpe = peb[r, pl.ds(o, _LANES)]
                    for b in range(_B):
                        bufs[b][r, pl.ds(o, _LANES)] = (
                            bufs[b][r, pl.ds(o, _LANES)] + vpe
                        )
                return 0

            lax.fori_loop(0, _R * D // (_LANES * 8), add_body, 0)

        def block_body(gg, _):
            for gp in range(2):
                g = gg * 2 + gp
                par = gp

                pltpu.make_async_copy(pe_slice(g), pebufs[par], psems[par]).wait()
                wait_gathers(g, par)

                if gp == 0:
                    @pl.when(gg > 0)
                    def _():
                        wait_scatters(g - 1, 1)
                    issue_gathers(g + 1, 1)
                else:
                    @pl.when(gg < n_blocks - 1)
                    def _():
                        wait_scatters(g - 1, 0)
                        issue_gathers(g + 1, 0)

                add_chunk(par)

                @pl.when(gg < n_blocks - 1)
                def _():
                    pltpu.async_copy(pe_slice(g + 2), pebufs[par], psems[par])

                issue_scatters(g, par)
            return 0

        lax.fori_loop(0, n_blocks, block_body, 0)

        wait_scatters(n_chunks - 2, 0)
        wait_scatters(n_chunks - 1, 1)

    return sc_add(x, pe)


# speedup vs baseline: 3.8339x; 1.0621x over previous
import functools

import jax
import jax.numpy as jnp
from jax import lax
from jax.experimental import pallas as pl
from jax.experimental.pallas import tpu as pltpu
from jax.experimental.pallas import tpu_sc as plsc

_R = 8
_NW = 32
_LANES = 16
_B = 4


def kernel(x, pe_table):
    B, S, D = x.shape
    rows_per_w = S // _NW
    n_chunks = rows_per_w // _R
    n_blocks = n_chunks // 2

    pe = pe_table[:S]

    mesh = plsc.VectorSubcoreMesh(core_axis_name="c", subcore_axis_name="s")

    vmem = lambda: pltpu.VMEM((_R, D), jnp.float32)
    sem = pltpu.SemaphoreType.DMA

    @functools.partial(
        pl.kernel,
        mesh=mesh,
        out_type=jax.ShapeDtypeStruct((B, S, D), jnp.float32),
        scratch_types=(
            [vmem() for _ in range(2)]
            + [vmem() for _ in range(2 * _B)]
            + [sem] * 6
        ),
        compiler_params=pltpu.CompilerParams(use_tc_tiling_on_sc=True),
    )
    def sc_add(x_hbm, pe_hbm, out_hbm, pb0, pb1,
               xa0, xa1, xa2, xa3, xb0, xb1, xb2, xb3,
               ps0, ps1, gs0, gs1, ss0, ss1):
        pebufs = (pb0, pb1)
        xbufs = ((xa0, xa1, xa2, xa3), (xb0, xb1, xb2, xb3))
        psems = (ps0, ps1)
        gsems = (gs0, gs1)
        ssems = (ss0, ss1)

        wid = lax.axis_index("s") * 2 + lax.axis_index("c")
        base = wid * rows_per_w

        def pe_slice(g):
            return pe_hbm.at[pl.ds(base + g * _R, _R)]

        def x_slice(g, b):
            return x_hbm.at[b, pl.ds(base + g * _R, _R)]

        def out_slice(g, b):
            return out_hbm.at[b, pl.ds(base + g * _R, _R)]

        def issue_gathers(g, par):
            for b in range(_B):
                pltpu.async_copy(x_slice(g, b), xbufs[par][b], gsems[par])

        def wait_gathers(g, par):
            for b in range(_B):
                pltpu.make_async_copy(
                    x_slice(g, b), xbufs[par][b], gsems[par]).wait()

        def wait_scatters(g, par):
            for b in range(_B):
                pltpu.make_async_copy(
                    out_slice(g, b), xbufs[par][b], ssems[par]).wait()

        def issue_scatters(g, par):
            for b in range(_B):
                pltpu.async_copy(xbufs[par][b], out_slice(g, b), ssems[par])

        pltpu.async_copy(pe_slice(0), pb0, ps0)
        pltpu.async_copy(pe_slice(1), pb1, ps1)
        issue_gathers(0, 0)

        def add_chunk(par):
            bufs = xbufs[par]
            peb = pebufs[par]

            def add_body(i, _):
                r = i >> 4
                cb = (i & 15) * (D // 16)
                for k in range(4):
                    o = cb + k * _LANES
                    vpe = peb[r, pl.ds(o, _LANES)]
                    for b in range(_B):
                        bufs[b][r, pl.ds(o, _LANES)] = (
                            bufs[b][r, pl.ds(o, _LANES)] + vpe
                        )
                return 0

            lax.fori_loop(0, _R * D // (_LANES * 4), add_body, 0)

        def block_body(gg, _):
            for gp in range(2):
                g = gg * 2 + gp
                par = gp

                pltpu.make_async_copy(pe_slice(g), pebufs[par], psems[par]).wait()
                wait_gathers(g, par)

                if gp == 0:
                    @pl.when(gg > 0)
                    def _():
                        wait_scatters(g - 1, 1)
                    issue_gathers(g + 1, 1)
                else:
                    @pl.when(gg < n_blocks - 1)
                    def _():
                        wait_scatters(g - 1, 0)
                        issue_gathers(g + 1, 0)

                add_chunk(par)

                @pl.when(gg < n_blocks - 1)
                def _():
                    pltpu.async_copy(pe_slice(g + 2), pebufs[par], psems[par])

                issue_scatters(g, par)
            return 0

        lax.fori_loop(0, n_blocks, block_body, 0)

        wait_scatters(n_chunks - 2, 0)
        wait_scatters(n_chunks - 1, 1)

    return sc_add(x, pe)


# speedup vs baseline: 3.8360x; 1.0006x over previous
import functools

import jax
import jax.numpy as jnp
from jax import lax
from jax.experimental import pallas as pl
from jax.experimental.pallas import tpu as pltpu
from jax.experimental.pallas import tpu_sc as plsc

_R = 8
_NW = 32
_LANES = 16
_B = 4
_P = 3


def kernel(x, pe_table):
    B, S, D = x.shape
    rows_per_w = S // _NW
    n_chunks = rows_per_w // _R
    n_blocks = (n_chunks - 2) // _P

    pe = pe_table[:S]

    mesh = plsc.VectorSubcoreMesh(core_axis_name="c", subcore_axis_name="s")

    vmem = lambda: pltpu.VMEM((_R, D), jnp.float32)
    sem = pltpu.SemaphoreType.DMA

    @functools.partial(
        pl.kernel,
        mesh=mesh,
        out_type=jax.ShapeDtypeStruct((B, S, D), jnp.float32),
        scratch_types=(
            [vmem() for _ in range(_P)]
            + [vmem() for _ in range(_P * _B)]
            + [sem] * (3 * _P)
        ),
        compiler_params=pltpu.CompilerParams(use_tc_tiling_on_sc=True),
    )
    def sc_add(x_hbm, pe_hbm, out_hbm, pb0, pb1, pb2,
               xa0, xa1, xa2, xa3, xb0, xb1, xb2, xb3, xc0, xc1, xc2, xc3,
               ps0, ps1, ps2, gs0, gs1, gs2, ss0, ss1, ss2):
        pebufs = (pb0, pb1, pb2)
        xbufs = ((xa0, xa1, xa2, xa3), (xb0, xb1, xb2, xb3),
                 (xc0, xc1, xc2, xc3))
        psems = (ps0, ps1, ps2)
        gsems = (gs0, gs1, gs2)
        ssems = (ss0, ss1, ss2)

        wid = lax.axis_index("s") * 2 + lax.axis_index("c")
        base = wid * rows_per_w

        def pe_slice(g):
            return pe_hbm.at[pl.ds(base + g * _R, _R)]

        def x_slice(g, b):
            return x_hbm.at[b, pl.ds(base + g * _R, _R)]

        def out_slice(g, b):
            return out_hbm.at[b, pl.ds(base + g * _R, _R)]

        def issue_gathers(g, par):
            for b in range(_B):
                pltpu.async_copy(x_slice(g, b), xbufs[par][b], gsems[par])

        def wait_gathers(g, par):
            for b in range(_B):
                pltpu.make_async_copy(
                    x_slice(g, b), xbufs[par][b], gsems[par]).wait()

        def wait_scatters(g, par):
            for b in range(_B):
                pltpu.make_async_copy(
                    out_slice(g, b), xbufs[par][b], ssems[par]).wait()

        def issue_scatters(g, par):
            for b in range(_B):
                pltpu.async_copy(xbufs[par][b], out_slice(g, b), ssems[par])

        def add_chunk(par):
            bufs = xbufs[par]
            peb = pebufs[par]

            def add_body(i, _):
                r = i >> 4
                cb = (i & 15) * (D // 16)
                for k in range(4):
                    o = cb + k * _LANES
                    vpe = peb[r, pl.ds(o, _LANES)]
                    for b in range(_B):
                        bufs[b][r, pl.ds(o, _LANES)] = (
                            bufs[b][r, pl.ds(o, _LANES)] + vpe
                        )
                return 0

            lax.fori_loop(0, _R * D // (_LANES * 4), add_body, 0)

        def chunk_step(g, par, first, pe_pred, gather_pred):
            npar = (par + 2) % _P
            pltpu.make_async_copy(pe_slice(g), pebufs[par], psems[par]).wait()
            wait_gathers(g, par)
            add_chunk(par)
            issue_scatters(g, par)
            if not first:
                wait_scatters(g - 1, npar)
            if gather_pred is True:
                issue_gathers(g + 2, npar)
            elif gather_pred is not None:
                @pl.when(gather_pred)
                def _():
                    issue_gathers(g + 2, npar)
            if pe_pred is True:
                pltpu.async_copy(pe_slice(g + _P), pebufs[par], psems[par])
            elif pe_pred is not None:
                @pl.when(pe_pred)
                def _():
                    pltpu.async_copy(pe_slice(g + _P), pebufs[par], psems[par])

        pltpu.async_copy(pe_slice(0), pb0, ps0)
        pltpu.async_copy(pe_slice(1), pb1, ps1)
        pltpu.async_copy(pe_slice(2), pb2, ps2)
        issue_gathers(0, 0)
        issue_gathers(1, 1)

        chunk_step(0, 0, first=True, pe_pred=True, gather_pred=True)
        chunk_step(1, 1, first=False, pe_pred=True, gather_pred=True)

        def block_body(gg, _):
            for j in range(_P):
                g = gg * _P + 2 + j
                par = (2 + j) % _P
                last_block = gg < n_blocks - 1
                gather_pred = True if j == 0 else last_block
                chunk_step(g, par, first=False,
                           pe_pred=last_block, gather_pred=gather_pred)
            return 0

        lax.fori_loop(0, n_blocks, block_body, 0)

        wait_scatters(n_chunks - 1, (n_chunks - 1) % _P)

    return sc_add(x, pe)
